# baseline (speedup 1.0000x reference)
import functools

import jax
import jax.numpy as jnp
from jax import lax
from jax.experimental import pallas as pl
from jax.experimental.pallas import tpu as pltpu
from jax.experimental.pallas import tpu_sc as plsc

B = 4096
F = 26
D = 18
DP = 32
IN_DIM = F * D
VOCAB = 1000000
ROWS_PER_LINE = 128 // DP
LINES = VOCAB // ROWS_PER_LINE

NC = 2
NS = 16
NW = NC * NS
TOTAL = B * F
PER_W = TOTAL // NW
CHUNK = 128
NCHUNK = PER_W // CHUNK
OUT_SUB = CHUNK * D // 128
EXT_SUB = 24
W_LINES = PER_W * D // 128

CBR = 5000

BLK = 512
GRID = B // BLK


def _compact_body(t0, t1, t2, t3, out_ref):
    for a, t in enumerate((t0, t1, t2, t3)):
        out_ref[:, pl.ds(a * DP, D)] = t[...]


def _compact(emb_table):
    def spec(a):
        return pl.BlockSpec((CBR, D), lambda i, _a=a: (_a * (LINES // CBR) + i, 0))
    return pl.pallas_call(
        _compact_body,
        grid=(LINES // CBR,),
        in_specs=[spec(a) for a in range(ROWS_PER_LINE)],
        out_specs=pl.BlockSpec((CBR, 128), lambda i: (i, 0)),
        out_shape=jax.ShapeDtypeStruct((LINES, 128), jnp.float32),
    )(emb_table, emb_table, emb_table, emb_table)


def _gather_body(tab128, lines, ext, out, line_v, win_v, ext_v, out_v,
                 sw0, sw1, se0, se1):
    wid = lax.axis_index("s") * NC + lax.axis_index("c")
    sw = (sw0, sw1)
    se = (se0, se1)
    pltpu.sync_copy(lines.at[wid], line_v)
    cps = {}

    def fire(c):
        s = c & 1
        cps[c] = (
            pltpu.async_copy(tab128.at[line_v.at[c]],
                             win_v.at[pl.ds(s * CHUNK, CHUNK)], sw[s]),
            pltpu.async_copy(ext.at[wid * NCHUNK + c],
                             ext_v.at[pl.ds(s * EXT_SUB, EXT_SUB)], se[s]),
        )

    def drain_extract(c):
        s = c & 1
        cps[c][0].wait()
        cps[c][1].wait()
        base = c * OUT_SUB
        ebase = s * EXT_SUB
        wbase = s * CHUNK

        def ext_step(t, _):
            r = t >> 3
            g = (t & 7) * 16
            sv = ext_v[ebase + r, pl.ds(g, 16)]
            out_v[base + r, pl.ds(g, 16)] = plsc.load_gather(
                win_v, [(sv >> 7) + wbase, sv & 127])
            return 0

        lax.fori_loop(0, OUT_SUB * 8, ext_step, 0)

    fire(0)
    for c in range(1, NCHUNK):
        fire(c)
        drain_extract(c - 1)
    drain_extract(NCHUNK - 1)
    pltpu.sync_copy(out_v, out.at[wid])


def _sc_gather(tab128, x):
    x_flat = x.reshape(TOTAL)
    lines_idx = (x_flat % LINES).reshape(NW, NCHUNK, CHUNK)
    lines_idx = jnp.pad(lines_idx, ((0, 0), (0, 32 - NCHUNK), (0, 0)))
    off = (x_flat // LINES) * DP
    src = off[:, None] + jnp.arange(D, dtype=jnp.int32)[None, :]
    src = src.reshape(NW * NCHUNK, CHUNK, D)
    src = src + (jnp.arange(CHUNK, dtype=jnp.int32) * 128)[None, :, None]
    ext = src.reshape(NW * NCHUNK, OUT_SUB, 128)
    ext = jnp.pad(ext, ((0, 0), (0, EXT_SUB - OUT_SUB), (0, 0)))

    mesh = plsc.VectorSubcoreMesh(core_axis_name="c", subcore_axis_name="s")
    fn = functools.partial(
        pl.kernel,
        mesh=mesh,
        out_type=jax.ShapeDtypeStruct((NW, W_LINES, 128), jnp.float32),
        scratch_types=[
            pltpu.VMEM((32, CHUNK), jnp.int32),
            pltpu.VMEM((2 * CHUNK, 128), jnp.float32),
            pltpu.VMEM((2 * EXT_SUB, 128), jnp.int32),
            pltpu.VMEM((W_LINES, 128), jnp.float32),
            pltpu.SemaphoreType.DMA,
            pltpu.SemaphoreType.DMA,
            pltpu.SemaphoreType.DMA,
            pltpu.SemaphoreType.DMA,
        ],
        compiler_params=pltpu.CompilerParams(needs_layout_passes=False),
    )(_gather_body)
    return fn(tab128, lines_idx, ext)


def _mlp_body(feat_ref,
              cw0, cb0, cw1, cb1, cw2, cb2, cw3, cb3, cw4, cb4,
              vw0, vb0, vw1, vb1, vw2, vb2, vw3, vb3, vw4, vb4,
              ctr_out, cvr_out):
    h = feat_ref[...]

    def tower(ws, bs):
        a = h
        for i in range(4):
            a = jnp.maximum(
                jnp.dot(a, ws[i][...], preferred_element_type=jnp.float32)
                + bs[i][...], 0.0)
        a = jnp.dot(a, ws[4][...], preferred_element_type=jnp.float32) + bs[4][...]
        return 1.0 / (1.0 + jnp.exp(-a))

    ctr_out[...] = tower((cw0, cw1, cw2, cw3, cw4), (cb0, cb1, cb2, cb3, cb4))
    cvr_out[...] = tower((vw0, vw1, vw2, vw3, vw4), (vb0, vb1, vb2, vb3, vb4))


def _mlp_call(feat, weights):
    full = lambda w: pl.BlockSpec(w.shape, lambda i, _nd=w.ndim: (0,) * _nd)
    in_specs = [pl.BlockSpec((BLK, IN_DIM), lambda i: (i, 0))]
    in_specs += [full(w) for w in weights]
    out_specs = [pl.BlockSpec((BLK, 1), lambda i: (i, 0))] * 2
    out_shape = [jax.ShapeDtypeStruct((B, 1), jnp.float32)] * 2
    return pl.pallas_call(
        _mlp_body,
        grid=(GRID,),
        in_specs=in_specs,
        out_specs=out_specs,
        out_shape=out_shape,
    )(feat, *weights)


def kernel(x, emb_table,
           ctr_W0, ctr_b0, ctr_W1, ctr_b1, ctr_W2, ctr_b2, ctr_W3, ctr_b3,
           ctr_W4, ctr_b4,
           cvr_W0, cvr_b0, cvr_W1, cvr_b1, cvr_W2, cvr_b2, cvr_W3, cvr_b3,
           cvr_W4, cvr_b4):
    tab128 = _compact(emb_table)
    rows = _sc_gather(tab128, x)
    feat = rows.reshape(B, IN_DIM)
    weights = (ctr_W0, ctr_b0, ctr_W1, ctr_b1, ctr_W2, ctr_b2, ctr_W3, ctr_b3,
               ctr_W4, ctr_b4,
               cvr_W0, cvr_b0, cvr_W1, cvr_b1, cvr_W2, cvr_b2, cvr_W3, cvr_b3,
               cvr_W4, cvr_b4)
    ctr, cvr = _mlp_call(feat, weights)
    return (ctr, cvr)


# speedup vs baseline: 1.0066x; 1.0066x over previous
import functools

import jax
import jax.numpy as jnp
from jax import lax
from jax.experimental import pallas as pl
from jax.experimental.pallas import tpu as pltpu
from jax.experimental.pallas import tpu_sc as plsc

B = 4096
F = 26
D = 18
DP = 32
IN_DIM = F * D
VOCAB = 1000000
ROWS_PER_LINE = 128 // DP
LINES = VOCAB // ROWS_PER_LINE

NC = 2
NS = 16
NW = NC * NS
TOTAL = B * F
PER_W = TOTAL // NW
CHUNK = 128
NCHUNK = PER_W // CHUNK
OUT_SUB = CHUNK * D // 128
EXT_SUB = 24
W_LINES = PER_W * D // 128

CBR = 5000

BLK = 1024
GRID = B // BLK


def _compact_body(t0, t1, t2, t3, out_ref):
    for a, t in enumerate((t0, t1, t2, t3)):
        out_ref[:, pl.ds(a * DP, D)] = t[...]


def _compact(emb_table):
    def spec(a):
        return pl.BlockSpec((CBR, D), lambda i, _a=a: (_a * (LINES // CBR) + i, 0))
    return pl.pallas_call(
        _compact_body,
        grid=(LINES // CBR,),
        in_specs=[spec(a) for a in range(ROWS_PER_LINE)],
        out_specs=pl.BlockSpec((CBR, 128), lambda i: (i, 0)),
        out_shape=jax.ShapeDtypeStruct((LINES, 128), jnp.float32),
    )(emb_table, emb_table, emb_table, emb_table)


def _gather_body(tab128, lines, ext, out, line_v, win_v, ext_v, out_v,
                 sw0, sw1, se0, se1):
    wid = lax.axis_index("s") * NC + lax.axis_index("c")
    sw = (sw0, sw1)
    se = (se0, se1)
    pltpu.sync_copy(lines.at[wid], line_v)
    cps = {}

    def fire(c):
        s = c & 1
        cps[c] = (
            pltpu.async_copy(tab128.at[line_v.at[c]],
                             win_v.at[pl.ds(s * CHUNK, CHUNK)], sw[s]),
            pltpu.async_copy(ext.at[wid * NCHUNK + c],
                             ext_v.at[pl.ds(s * EXT_SUB, EXT_SUB)], se[s]),
        )

    def drain_extract(c):
        s = c & 1
        cps[c][0].wait()
        cps[c][1].wait()
        base = c * OUT_SUB
        ebase = s * EXT_SUB
        wbase = s * CHUNK

        def ext_step(t, _):
            r = t >> 3
            g = (t & 7) * 16
            sv = ext_v[ebase + r, pl.ds(g, 16)]
            out_v[base + r, pl.ds(g, 16)] = plsc.load_gather(
                win_v, [(sv >> 7) + wbase, sv & 127])
            return 0

        lax.fori_loop(0, OUT_SUB * 8, ext_step, 0)

    fire(0)
    for c in range(1, NCHUNK):
        fire(c)
        drain_extract(c - 1)
    drain_extract(NCHUNK - 1)
    pltpu.sync_copy(out_v, out.at[wid])


def _sc_gather(tab128, x):
    x_flat = x.reshape(TOTAL)
    lines_idx = (x_flat % LINES).reshape(NW, NCHUNK, CHUNK)
    lines_idx = jnp.pad(lines_idx, ((0, 0), (0, 32 - NCHUNK), (0, 0)))
    off = (x_flat // LINES) * DP
    src = off[:, None] + jnp.arange(D, dtype=jnp.int32)[None, :]
    src = src.reshape(NW * NCHUNK, CHUNK, D)
    src = src + (jnp.arange(CHUNK, dtype=jnp.int32) * 128)[None, :, None]
    ext = src.reshape(NW * NCHUNK, OUT_SUB, 128)
    ext = jnp.pad(ext, ((0, 0), (0, EXT_SUB - OUT_SUB), (0, 0)))

    mesh = plsc.VectorSubcoreMesh(core_axis_name="c", subcore_axis_name="s")
    fn = functools.partial(
        pl.kernel,
        mesh=mesh,
        out_type=jax.ShapeDtypeStruct((NW, W_LINES, 128), jnp.float32),
        scratch_types=[
            pltpu.VMEM((32, CHUNK), jnp.int32),
            pltpu.VMEM((2 * CHUNK, 128), jnp.float32),
            pltpu.VMEM((2 * EXT_SUB, 128), jnp.int32),
            pltpu.VMEM((W_LINES, 128), jnp.float32),
            pltpu.SemaphoreType.DMA,
            pltpu.SemaphoreType.DMA,
            pltpu.SemaphoreType.DMA,
            pltpu.SemaphoreType.DMA,
        ],
        compiler_params=pltpu.CompilerParams(needs_layout_passes=False),
    )(_gather_body)
    return fn(tab128, lines_idx, ext)


def _mlp_body(feat_ref,
              cw0, cb0, cw1, cb1, cw2, cb2, cw3, cb3, cw4, cb4,
              vw0, vb0, vw1, vb1, vw2, vb2, vw3, vb3, vw4, vb4,
              ctr_out, cvr_out):
    h = feat_ref[...]

    def tower(ws, bs):
        a = h
        for i in range(4):
            a = jnp.maximum(
                jnp.dot(a, ws[i][...], preferred_element_type=jnp.float32)
                + bs[i][...], 0.0)
        a = jnp.dot(a, ws[4][...], preferred_element_type=jnp.float32) + bs[4][...]
        return 1.0 / (1.0 + jnp.exp(-a))

    ctr_out[...] = tower((cw0, cw1, cw2, cw3, cw4), (cb0, cb1, cb2, cb3, cb4))
    cvr_out[...] = tower((vw0, vw1, vw2, vw3, vw4), (vb0, vb1, vb2, vb3, vb4))


def _mlp_call(feat, weights):
    full = lambda w: pl.BlockSpec(w.shape, lambda i, _nd=w.ndim: (0,) * _nd)
    in_specs = [pl.BlockSpec((BLK, IN_DIM), lambda i: (i, 0))]
    in_specs += [full(w) for w in weights]
    out_specs = [pl.BlockSpec((BLK, 1), lambda i: (i, 0))] * 2
    out_shape = [jax.ShapeDtypeStruct((B, 1), jnp.float32)] * 2
    return pl.pallas_call(
        _mlp_body,
        grid=(GRID,),
        in_specs=in_specs,
        out_specs=out_specs,
        out_shape=out_shape,
    )(feat, *weights)


def kernel(x, emb_table,
           ctr_W0, ctr_b0, ctr_W1, ctr_b1, ctr_W2, ctr_b2, ctr_W3, ctr_b3,
           ctr_W4, ctr_b4,
           cvr_W0, cvr_b0, cvr_W1, cvr_b1, cvr_W2, cvr_b2, cvr_W3, cvr_b3,
           cvr_W4, cvr_b4):
    tab128 = _compact(emb_table)
    rows = _sc_gather(tab128, x)
    feat = rows.reshape(B, IN_DIM)
    weights = (ctr_W0, ctr_b0, ctr_W1, ctr_b1, ctr_W2, ctr_b2, ctr_W3, ctr_b3,
               ctr_W4, ctr_b4,
               cvr_W0, cvr_b0, cvr_W1, cvr_b1, cvr_W2, cvr_b2, cvr_W3, cvr_b3,
               cvr_W4, cvr_b4)
    ctr, cvr = _mlp_call(feat, weights)
    return (ctr, cvr)
